# baseline (speedup 1.0000x reference)
import functools

import jax
import jax.numpy as jnp
from jax import lax
from jax.experimental import pallas as pl
from jax.experimental.pallas import tpu as pltpu
from jax.experimental.pallas import tpu_sc as plsc

NI = 4096
NJ = 200
EMBED = 64
NC, NS = 2, 16
NW = NC * NS
DB, DS = 8, 8
IB, IL = NI // 128, 128
SLABS = NJ * DB
PER_W = SLABS // NW
RING = 2
L = 16


def _sc_body(idst_hbm, tables_hbm, out_hbm, ids_v, table_v, buf_v, ssem):
    wid = lax.axis_index("s") * NC + lax.axis_index("c")
    base_w = wid * PER_W

    pltpu.sync_copy(tables_hbm, table_v)

    def store(j, db, b):
        return pltpu.make_async_copy(
            buf_v.at[b],
            out_hbm.at[j, db],
            ssem,
        )

    def slab(t, b):
        s = base_w + t
        j = s // DB
        db = s % DB

        @pl.when(jnp.logical_or(t == 0, db == 0))
        def _stage_ids():
            pltpu.sync_copy(idst_hbm.at[j], ids_v)

        t0s = [table_v[0, db * DS + ds, :] for ds in range(DS)]
        t1s = [table_v[1, db * DS + ds, :] for ds in range(DS)]
        t2s = [table_v[2, db * DS + ds, :] for ds in range(DS)]

        @plsc.parallel_loop(0, IB, unroll=4)
        def ib_body(ib):
            for g in range(IL // L):
                ids_g = ids_v[pl.ds(ib * IL + g * L, L)]
                m1 = ids_g == 1
                m2 = ids_g == 2
                for ds in range(DS):
                    v = jnp.where(m1, t1s[ds], jnp.where(m2, t2s[ds], t0s[ds]))
                    buf_v[b, ib, ds, pl.ds(g * L, L)] = v

        return j, db

    def group(gidx, carry):
        for b in range(RING):
            t = gidx * RING + b

            @pl.when(t >= RING)
            def _wait_buffer_free():
                store(0, 0, b).wait()

            j, db = slab(t, b)
            store(j, db, b).start()

        return carry

    lax.fori_loop(0, PER_W // RING, group, 0)
    store(0, 0, 0).wait()
    store(0, 0, 1).wait()


def kernel(modality_ids, modality_embedding):
    idst = modality_ids.astype(jnp.int32).T
    tables = jnp.tile(modality_embedding[:, :, None], (1, 1, L))

    mesh = plsc.VectorSubcoreMesh(core_axis_name="c", subcore_axis_name="s")
    run = functools.partial(
        pl.kernel,
        mesh=mesh,
        out_type=jax.ShapeDtypeStruct((NJ, DB, IB, DS, IL), jnp.float32),
        scratch_types=[
            pltpu.VMEM((NI,), jnp.int32),
            pltpu.VMEM((3, EMBED, L), jnp.float32),
            pltpu.VMEM((RING, IB, DS, IL), jnp.float32),
            pltpu.SemaphoreType.DMA,
        ],
    )(_sc_body)
    out5 = run(idst, tables)
    return out5.transpose(2, 4, 0, 1, 3).reshape(NI, NJ, EMBED)


# speedup vs baseline: 1.0002x; 1.0002x over previous
import functools

import jax
import jax.numpy as jnp
from jax import lax
from jax.experimental import pallas as pl
from jax.experimental.pallas import tpu as pltpu
from jax.experimental.pallas import tpu_sc as plsc

NI = 4096
NJ = 200
EMBED = 64
NC, NS = 2, 16
NW = NC * NS
DB, DS = 8, 8
IB, IL = NI // 128, 128
SLABS = NJ * DB
PER_W = SLABS // NW
RING = 2
L = 16


def _sc_body(idst_hbm, tables_hbm, out_hbm, ids_v, table_v, buf_v, ssem):
    wid = lax.axis_index("s") * NC + lax.axis_index("c")
    base_w = wid * PER_W

    pltpu.sync_copy(tables_hbm, table_v)

    def store(j, db, b):
        return pltpu.make_async_copy(
            buf_v.at[b],
            out_hbm.at[j, db],
            ssem,
        )

    def slab(t, b):
        s = base_w + t
        j = s // DB
        db = s % DB

        @pl.when(jnp.logical_or(t == 0, db == 0))
        def _stage_ids():
            pltpu.sync_copy(idst_hbm.at[j], ids_v)

        t0s = [table_v[0, db * DS + ds, :] for ds in range(DS)]
        t1s = [table_v[1, db * DS + ds, :] for ds in range(DS)]
        t2s = [table_v[2, db * DS + ds, :] for ds in range(DS)]

        @plsc.parallel_loop(0, IB, unroll=2)
        def ib_body(ib):
            for g in range(IL // L):
                ids_g = ids_v[pl.ds(ib * IL + g * L, L)]
                m1 = ids_g == 1
                m2 = ids_g == 2
                for ds in range(DS):
                    v = jnp.where(m1, t1s[ds], jnp.where(m2, t2s[ds], t0s[ds]))
                    buf_v[b, ib, ds, pl.ds(g * L, L)] = v

        return j, db

    def group(gidx, carry):
        for b in range(RING):
            t = gidx * RING + b

            @pl.when(t >= RING)
            def _wait_buffer_free():
                store(0, 0, b).wait()

            j, db = slab(t, b)
            store(j, db, b).start()

        return carry

    lax.fori_loop(0, PER_W // RING, group, 0)
    store(0, 0, 0).wait()
    store(0, 0, 1).wait()


def kernel(modality_ids, modality_embedding):
    idst = modality_ids.astype(jnp.int32).T
    tables = jnp.tile(modality_embedding[:, :, None], (1, 1, L))

    mesh = plsc.VectorSubcoreMesh(core_axis_name="c", subcore_axis_name="s")
    run = functools.partial(
        pl.kernel,
        mesh=mesh,
        out_type=jax.ShapeDtypeStruct((NJ, DB, IB, DS, IL), jnp.float32),
        scratch_types=[
            pltpu.VMEM((NI,), jnp.int32),
            pltpu.VMEM((3, EMBED, L), jnp.float32),
            pltpu.VMEM((RING, IB, DS, IL), jnp.float32),
            pltpu.SemaphoreType.DMA,
        ],
    )(_sc_body)
    out5 = run(idst, tables)
    return out5.transpose(2, 4, 0, 1, 3).reshape(NI, NJ, EMBED)
